# Initial kernel scaffold; baseline (speedup 1.0000x reference)
import jax
import jax.numpy as jnp
from jax.experimental import pallas as pl


def kernel(x, target):
    raise NotImplementedError("write your pallas kernel here")


# speedup vs baseline: 2.5278x; 2.5278x over previous
import functools
import math

import jax
import jax.numpy as jnp
import numpy as np
from jax import lax
from jax.experimental import pallas as pl
from jax.experimental.pallas import tpu as pltpu
from jax.experimental.pallas import tpu_sc as plsc

_SIZE = 32000
_PAD = 0
_SMOOTHING = 0.1
_CONF = 1.0 - _SMOOTHING
_N = 4096

_EPS = float(np.float32(_SMOOTHING / (_SIZE - 2)))
_K0 = (_SIZE - 2) * _EPS * math.log(_EPS) + _CONF * math.log(_CONF)

_NC, _NS, _L = 2, 16, 16
_NW = _NC * _NS
_BPW = _N // _NW

@functools.lru_cache(maxsize=None)
def _make_sc_gather():
    mesh = plsc.VectorSubcoreMesh(
        core_axis_name="c", subcore_axis_name="s", num_cores=_NC, num_subcores=_NS
    )

    @functools.partial(
        pl.kernel,
        out_type=jax.ShapeDtypeStruct((_N,), jnp.float32),
        mesh=mesh,
        scratch_types=[
            pltpu.VMEM((_BPW,), jnp.int32),
            pltpu.VMEM((_BPW,), jnp.int32),
            pltpu.VMEM((_BPW,), jnp.float32),
            pltpu.SemaphoreType.DMA,
        ],
    )
    def _sc_gather(xf_hbm, tgt_hbm, out_hbm, tgt_v, idx_v, g_v, sem):
        wid = lax.axis_index("s") * _NC + lax.axis_index("c")
        base = wid * _BPW
        pltpu.sync_copy(tgt_hbm.at[pl.ds(base, _BPW)], tgt_v)
        for k in range(_BPW // _L):
            row = base + k * _L + lax.iota(jnp.int32, _L)
            idx_v[pl.ds(k * _L, _L)] = row * _SIZE + tgt_v[pl.ds(k * _L, _L)]
        pltpu.async_copy(xf_hbm.at[idx_v], g_v, sem).wait()
        pltpu.sync_copy(g_v, out_hbm.at[pl.ds(base, _BPW)])

    return _sc_gather


_BC = 1280
_KC = _BC // 128
_NBLK = _SIZE // _BC


def _tc_body(x_ref, t_ref, g_ref, out_ref, acc_ref):
    j = pl.program_id(0)

    @pl.when(j == 0)
    def _init():
        acc_ref[...] = jnp.zeros_like(acc_ref)

    acc = acc_ref[...]
    for k in range(_KC):
        chunk = x_ref[:, k * 128:(k + 1) * 128]
        if k == 0:
            lane = lax.broadcasted_iota(jnp.int32, (_N, 128), 1)
            chunk = jnp.where((j == 0) & (lane == 0), 0.0, chunk)
        acc = acc + chunk
    acc_ref[...] = acc

    @pl.when(j == _NBLK - 1)
    def _finish():
        rowsum = jnp.sum(acc_ref[...], axis=1, keepdims=True)
        g = g_ref[...]
        valid = t_ref[...] != _PAD
        li = _K0 - _EPS * (rowsum - g) - _CONF * g
        out_ref[0, 0] = jnp.sum(jnp.where(valid, li, 0.0))


_tc_reduce = pl.pallas_call(
    _tc_body,
    grid=(_NBLK,),
    in_specs=[
        pl.BlockSpec((_N, _BC), lambda j: (0, j)),
        pl.BlockSpec((_N, 1), lambda j: (0, 0)),
        pl.BlockSpec((_N, 1), lambda j: (0, 0)),
    ],
    out_specs=pl.BlockSpec((1, 1), lambda j: (0, 0), memory_space=pltpu.SMEM),
    out_shape=jax.ShapeDtypeStruct((1, 1), jnp.float32),
    scratch_shapes=[pltpu.VMEM((_N, 128), jnp.float32)],
)


def kernel(x, target):
    g = _make_sc_gather()(x.reshape(-1), target)
    loss = _tc_reduce(x, target.reshape(_N, 1), g.reshape(_N, 1))
    return loss.reshape(())
